# baseline (speedup 1.0000x reference)
import functools

import jax
import jax.numpy as jnp
from jax import lax
from jax.experimental import pallas as pl
from jax.experimental.pallas import tpu as pltpu
from jax.experimental.pallas import tpu_sc as plsc

F32 = jnp.float32

_G = 32
_NSC = 2
_NSUB = 16
_NW = _NSC * _NSUB


def _cdiv(a, b):
    return (a + b - 1) // b


def _ee_body(a_ref, w1_ref, w2_ref, w3_ref, e1_ref, e2_ref, e3_ref):
    a = a_ref[:]
    e1_ref[:] = jnp.dot(a, w1_ref[:], preferred_element_type=F32)
    e2_ref[:] = jnp.dot(a, w2_ref[:], preferred_element_type=F32)
    e3_ref[:] = jnp.dot(a, w3_ref[:], preferred_element_type=F32)


def _proj_body(h_ref, wd_ref, bd_ref, ws_ref, pd_ref, ps_ref):
    h = h_ref[:]
    pd_ref[:] = jnp.dot(h, wd_ref[:], preferred_element_type=F32) + bd_ref[:]
    ps_ref[:] = jnp.dot(h, ws_ref[:], preferred_element_type=F32)


def _mid12_body(parts_ref, x_ref, lw_ref, lb_ref, wd_ref, bd_ref, ws_ref,
                h2_ref, pd_ref, ps_ref):
    h1r = jnp.maximum(parts_ref[0] + parts_ref[1] + x_ref[:], 0.0)
    h2 = jnp.dot(h1r, lw_ref[:], preferred_element_type=F32) + lb_ref[:]
    h2_ref[:] = h2
    pd_ref[:] = jnp.dot(h2, wd_ref[:], preferred_element_type=F32) + bd_ref[:]
    ps_ref[:] = jnp.dot(h2, ws_ref[:], preferred_element_type=F32)


def _mid23_body(parts_ref, h_ref, wd_ref, bd_ref, ws_ref,
                hr_ref, pd_ref, ps_ref):
    hr = jnp.maximum(parts_ref[0] + parts_ref[1] + h_ref[:], 0.0)
    hr_ref[:] = hr
    pd_ref[:] = jnp.dot(hr, wd_ref[:], preferred_element_type=F32) + bd_ref[:]
    ps_ref[:] = jnp.dot(hr, ws_ref[:], preferred_element_type=F32)


def _post_body(parts_ref, h_ref, o_ref):
    o_ref[:] = jnp.maximum(parts_ref[0] + parts_ref[1] + h_ref[:], 0.0)


def _full(shape):
    return pl.BlockSpec(shape, lambda i: tuple(0 for _ in shape))


def _rows(bm, w):
    return pl.BlockSpec((bm, w), lambda i: (i, 0))


def _parts_spec(bm, w):
    return pl.BlockSpec((_NSC, bm, w), lambda i: (0, i, 0))


def _softplus16(ts):
    m = jnp.maximum(ts, 0.0)
    w = jnp.exp(-jnp.abs(ts))
    s = w / (2.0 + w)
    s2 = s * s
    p = s * (2.0 + s2 * (2.0 / 3.0 + s2 * (2.0 / 5.0 + s2 * (
        2.0 / 7.0 + s2 * (2.0 / 9.0)))))
    return m + p


def _make_sc_edges(n_pad, c, t_chunks):
    mesh = plsc.VectorSubcoreMesh(core_axis_name="c", subcore_axis_name="s")
    rows_sub = n_pad // _NSUB
    c2 = 2 * c

    @functools.partial(
        pl.kernel,
        out_type=jax.ShapeDtypeStruct((_NSC, n_pad, c), F32),
        mesh=mesh,
        scratch_types=[
            pltpu.VMEM((_G,), jnp.int32),
            pltpu.VMEM((_G,), jnp.int32),
            pltpu.VMEM((_G, c2), F32),
            pltpu.VMEM((_G, c2), F32),
            pltpu.VMEM((_G, c2), F32),
            pltpu.VMEM((_G, c), F32),
            pltpu.VMEM_SHARED((n_pad, c), F32),
            pltpu.SemaphoreType.DMA,
            pltpu.SemaphoreType.DMA,
            pltpu.SemaphoreType.DMA,
        ],
    )
    def sc_edges(dst_hbm, src_hbm, pd_hbm, ps_hbm, ee_hbm, zero_hbm, out_hbm,
                 idxd, idxs, pdv, psv, eev, msgv, acc, sem1, sem2, sem3):
        cid = lax.axis_index("c")
        sid = lax.axis_index("s")
        wid = cid * _NSUB + sid

        row0 = pl.multiple_of(sid * rows_sub, 8)
        pltpu.sync_copy(zero_hbm.at[pl.ds(row0, rows_sub)],
                        acc.at[pl.ds(row0, rows_sub)])
        plsc.subcore_barrier()

        def chunk(tc, carry):
            base = pl.multiple_of((wid * t_chunks + tc) * _G, 8)
            pltpu.sync_copy(dst_hbm.at[pl.ds(base, _G)], idxd)
            pltpu.sync_copy(src_hbm.at[pl.ds(base, _G)], idxs)
            cp1 = pltpu.async_copy(pd_hbm.at[idxd], pdv, sem1)
            cp2 = pltpu.async_copy(ps_hbm.at[idxs], psv, sem2)
            cp3 = pltpu.async_copy(ee_hbm.at[pl.ds(base, _G)], eev, sem3)
            cp1.wait()
            cp2.wait()
            cp3.wait()

            def edge(e, carry2):
                for k in range(c // 16):
                    o = k * 16
                    tf = pdv[e, pl.ds(o, 16)] + psv[e, pl.ds(o, 16)] \
                        + eev[e, pl.ds(o, 16)]
                    ts = pdv[e, pl.ds(c + o, 16)] + psv[e, pl.ds(c + o, 16)] \
                        + eev[e, pl.ds(c + o, 16)]
                    gate = 1.0 / (1.0 + jnp.exp(-tf))
                    msgv[e, pl.ds(o, 16)] = gate * _softplus16(ts)
                return carry2

            lax.fori_loop(0, _G, edge, 0, unroll=False)
            pltpu.sync_copy(msgv, acc.at[idxd], add=True)
            return carry

        lax.fori_loop(0, t_chunks, chunk, 0, unroll=False)
        plsc.subcore_barrier()
        pltpu.sync_copy(acc.at[pl.ds(row0, rows_sub)],
                        out_hbm.at[cid, pl.ds(row0, rows_sub)])

    return sc_edges


def kernel(x, edge_index, edge_attr, conv1_Wf, conv1_bf, conv1_Ws, conv1_bs,
           lin_in_W, lin_in_b, c2_Wf, c2_bf, c2_Ws, c2_bs,
           c3_Wf, c3_bf, c3_Ws, c3_bs):
    n, c = x.shape
    e, d = edge_attr.shape
    c2 = 2 * c

    bm = 632
    n_pad = _cdiv(n + 8, bm) * bm
    t_chunks = _cdiv(e, _NW * _G)
    e_pad = _NW * _G * t_chunks

    def wsplit(wf, bf, ws, bs):
        wd = jnp.concatenate([wf[:c], ws[:c]], axis=1)
        bd = jnp.concatenate([bf, bs]).reshape(1, c2)
        wsr = jnp.concatenate([wf[c:c2], ws[c:c2]], axis=1)
        we = jnp.concatenate([wf[c2:], ws[c2:]], axis=1)
        return wd, bd, wsr, we

    wd1, bd1, wsr1, we1 = wsplit(conv1_Wf, conv1_bf, conv1_Ws, conv1_bs)
    wd2, bd2, wsr2, we2 = wsplit(c2_Wf, c2_bf, c2_Ws, c2_bs)
    wd3, bd3, wsr3, we3 = wsplit(c3_Wf, c3_bf, c3_Ws, c3_bs)

    pad_idx = (n + (jnp.arange(e_pad - e) % 8)).astype(jnp.int32)
    dst_p = jnp.concatenate([edge_index[1].astype(jnp.int32), pad_idx])
    src_p = jnp.concatenate([edge_index[0].astype(jnp.int32), pad_idx])
    attr_p = jnp.pad(edge_attr, ((0, e_pad - e), (0, 0)))
    x_p = jnp.pad(x, ((0, n_pad - n), (0, 0)))
    zeros = jnp.zeros((n_pad, c), F32)

    bme = 512
    ee1, ee2, ee3 = pl.pallas_call(
        _ee_body,
        grid=(e_pad // bme,),
        in_specs=[_rows(bme, d), _full((d, c2)), _full((d, c2)),
                  _full((d, c2))],
        out_specs=[_rows(bme, c2)] * 3,
        out_shape=[jax.ShapeDtypeStruct((e_pad, c2), F32)] * 3,
    )(attr_p, we1, we2, we3)

    grid_n = (n_pad // bm,)
    proj = pl.pallas_call(
        _proj_body,
        grid=grid_n,
        in_specs=[_rows(bm, c), _full((c, c2)), _full((1, c2)), _full((c, c2))],
        out_specs=[_rows(bm, c2)] * 2,
        out_shape=[jax.ShapeDtypeStruct((n_pad, c2), F32)] * 2,
    )
    pd1, ps1 = proj(x_p, wd1, bd1, wsr1)

    sc_edges = _make_sc_edges(n_pad, c, t_chunks)
    parts1 = sc_edges(dst_p, src_p, pd1, ps1, ee1, zeros)

    h2in, pd2, ps2 = pl.pallas_call(
        _mid12_body,
        grid=grid_n,
        in_specs=[_parts_spec(bm, c), _rows(bm, c), _full((c, c)),
                  _full((1, c)), _full((c, c2)), _full((1, c2)),
                  _full((c, c2))],
        out_specs=[_rows(bm, c), _rows(bm, c2), _rows(bm, c2)],
        out_shape=[jax.ShapeDtypeStruct((n_pad, c), F32),
                   jax.ShapeDtypeStruct((n_pad, c2), F32),
                   jax.ShapeDtypeStruct((n_pad, c2), F32)],
    )(parts1, x_p, lin_in_W, lin_in_b.reshape(1, c), wd2, bd2, wsr2)

    parts2 = sc_edges(dst_p, src_p, pd2, ps2, ee2, zeros)

    h2r, pd3, ps3 = pl.pallas_call(
        _mid23_body,
        grid=grid_n,
        in_specs=[_parts_spec(bm, c), _rows(bm, c), _full((c, c2)),
                  _full((1, c2)), _full((c, c2))],
        out_specs=[_rows(bm, c), _rows(bm, c2), _rows(bm, c2)],
        out_shape=[jax.ShapeDtypeStruct((n_pad, c), F32),
                   jax.ShapeDtypeStruct((n_pad, c2), F32),
                   jax.ShapeDtypeStruct((n_pad, c2), F32)],
    )(parts2, h2in, wd3, bd3, wsr3)

    parts3 = sc_edges(dst_p, src_p, pd3, ps3, ee3, zeros)

    out_full = pl.pallas_call(
        _post_body,
        grid=grid_n,
        in_specs=[_parts_spec(bm, c), _rows(bm, c)],
        out_specs=_rows(bm, c),
        out_shape=jax.ShapeDtypeStruct((n_pad, c), F32),
    )(parts3, h2r)

    return out_full[:n]
